# Initial kernel scaffold; baseline (speedup 1.0000x reference)
import jax
import jax.numpy as jnp
from jax.experimental import pallas as pl


def kernel(predicted_log_probabilities, tgt_tokens):
    raise NotImplementedError("write your pallas kernel here")


# speedup vs baseline: 2.7609x; 2.7609x over previous
import functools
import math

import jax
import jax.numpy as jnp
from jax import lax
from jax.experimental import pallas as pl
from jax.experimental.pallas import tpu as pltpu
from jax.experimental.pallas import tpu_sc as plsc

N = 4096
V = 32768
SMOOTH = 0.1
CONF = 1.0 - SMOOTH
R = SMOOTH / (V - 2)
K = (V - 2) * R * math.log(R) + CONF * math.log(CONF)

NC = 2
NS = 16
L = 16
NW = NC * NS
RPW = N // NW

CB = 1024


def _make_sc_gather():
    mesh = plsc.VectorSubcoreMesh(core_axis_name="c", subcore_axis_name="s")

    @functools.partial(
        pl.kernel,
        mesh=mesh,
        out_type=jax.ShapeDtypeStruct((NW, L), jnp.float32),
        scratch_types=[
            pltpu.VMEM((RPW,), jnp.int32),
            pltpu.VMEM((RPW,), jnp.int32),
            pltpu.VMEM((RPW,), jnp.float32),
            pltpu.VMEM((L,), jnp.float32),
            pltpu.SemaphoreType.DMA,
        ],
    )
    def sc_gather(pflat_hbm, tgt_hbm, out_hbm, tgt_v, fi_v, vals_v, acc_v, sem):
        wid = lax.axis_index("s") * NC + lax.axis_index("c")
        base = wid * RPW
        pltpu.sync_copy(tgt_hbm.at[pl.ds(base, RPW)], tgt_v)
        for j in range(RPW // L):
            t = tgt_v[pl.ds(j * L, L)]
            row = base + j * L + lax.iota(jnp.int32, L)
            fi_v[pl.ds(j * L, L)] = row * V + t
        pltpu.async_copy(pflat_hbm.at[fi_v], vals_v, sem).wait()
        acc = jnp.zeros((L,), jnp.float32)
        for j in range(RPW // L):
            t = tgt_v[pl.ds(j * L, L)]
            g = vals_v[pl.ds(j * L, L)]
            acc = acc + jnp.where(t != 0, K + (R - CONF) * g, 0.0)
        acc_v[...] = acc
        pltpu.sync_copy(acc_v, out_hbm.at[wid])

    return sc_gather


def _tc_body(p_ref, tgt_ref, scp_ref, out_ref):
    j = pl.program_id(0)
    mask = tgt_ref[...] != 0
    rowsums = jnp.sum(p_ref[...], axis=1, keepdims=True)
    bsum = jnp.sum(jnp.where(mask, rowsums, 0.0))

    @pl.when(j == 0)
    def _():
        col0 = jnp.sum(jnp.where(mask, p_ref[:, 0:1], 0.0))
        out_ref[0, 0] = jnp.sum(scp_ref[...]) + R * col0 - R * bsum

    @pl.when(j > 0)
    def _():
        out_ref[0, 0] = out_ref[0, 0] - R * bsum


def kernel(predicted_log_probabilities, tgt_tokens):
    p = predicted_log_probabilities
    pflat = p.reshape(N * V)
    scp = _make_sc_gather()(pflat, tgt_tokens)
    tgt2d = tgt_tokens.reshape(N, 1)
    out = pl.pallas_call(
        _tc_body,
        grid=(V // CB,),
        in_specs=[
            pl.BlockSpec((N, CB), lambda j: (0, j)),
            pl.BlockSpec((N, 1), lambda j: (0, 0)),
            pl.BlockSpec((NW, L), lambda j: (0, 0)),
        ],
        out_specs=pl.BlockSpec(memory_space=pltpu.SMEM),
        out_shape=jax.ShapeDtypeStruct((1, 1), jnp.float32),
    )(p, tgt2d, scp)
    return out[0, 0]


# speedup vs baseline: 7.9264x; 2.8709x over previous
import functools
import math

import jax
import jax.numpy as jnp
from jax import lax
from jax.experimental import pallas as pl
from jax.experimental.pallas import tpu as pltpu
from jax.experimental.pallas import tpu_sc as plsc

N = 4096
V = 32768
SMOOTH = 0.1
CONF = 1.0 - SMOOTH
R = SMOOTH / (V - 2)
K = (V - 2) * R * math.log(R) + CONF * math.log(CONF)

NC = 2
NS = 16
L = 16
NW = NC * NS
RPW = N // NW

G = 16
CB = 1024


def _make_sc_gather():
    mesh = plsc.VectorSubcoreMesh(core_axis_name="c", subcore_axis_name="s")

    @functools.partial(
        pl.kernel,
        mesh=mesh,
        out_type=jax.ShapeDtypeStruct((NW, L), jnp.float32),
        scratch_types=[
            pltpu.SMEM((G,), jnp.int32),
            pltpu.VMEM((RPW,), jnp.float32),
            pltpu.VMEM((G, 8, 128), jnp.float32),
            pltpu.VMEM((L,), jnp.float32),
            pltpu.VMEM((L,), jnp.float32),
            pltpu.SemaphoreType.DMA,
        ],
    )
    def sc_gather(p_hbm, tgt_hbm, out_hbm, tgt_s, tgt_v, tiles_v, mtmp_v,
                  acc_v, sem):
        wid = lax.axis_index("s") * NC + lax.axis_index("c")
        base = wid * RPW
        pltpu.sync_copy(tgt_hbm.at[pl.ds(base, RPW)], tgt_v)
        iota = lax.iota(jnp.int32, L)

        def chunk(g, acc):
            g0 = pl.multiple_of(g * G, G)
            tv = tgt_v[pl.ds(g0, G)]
            for k in range(G):
                tgt_s[k] = tv[k].astype(jnp.int32)
            ts = [tgt_s[k] for k in range(G)]
            copies = []
            for k in range(G):
                t = ts[k]
                r0 = pl.multiple_of(base + g0 + (k & ~7), 8)
                c0 = pl.multiple_of((t >> 7) << 7, 128)
                copies.append(
                    pltpu.async_copy(
                        p_hbm.at[pl.ds(r0, 8), pl.ds(c0, 128)],
                        tiles_v.at[k], sem))
            for c in copies:
                c.wait()
            for k in range(G):
                t = ts[k]
                cl = t & 0x70
                vec = tiles_v[k, k & 7, pl.ds(cl, L)]
                lane = jnp.where(t != 0, t & (L - 1), -1)
                acc = acc + jnp.where(iota == lane,
                                      (R - CONF) * vec + K, 0.0)
            return acc

        acc = lax.fori_loop(0, RPW // G, chunk, jnp.zeros((L,), jnp.float32))
        acc_v[...] = acc
        pltpu.sync_copy(acc_v, out_hbm.at[wid])

    return sc_gather


def _tc_body(p_ref, tgt_ref, scp_ref, out_ref):
    j = pl.program_id(0)
    mask = tgt_ref[...] != 0
    rowsums = jnp.sum(p_ref[...], axis=1, keepdims=True)
    bsum = jnp.sum(jnp.where(mask, rowsums, 0.0))

    @pl.when(j == 0)
    def _():
        col0 = jnp.sum(jnp.where(mask, p_ref[:, 0:1], 0.0))
        out_ref[0, 0] = jnp.sum(scp_ref[...]) + R * col0 - R * bsum

    @pl.when(j > 0)
    def _():
        out_ref[0, 0] = out_ref[0, 0] - R * bsum


def kernel(predicted_log_probabilities, tgt_tokens):
    p = predicted_log_probabilities
    scp = _make_sc_gather()(p, tgt_tokens.astype(jnp.float32))
    tgt2d = tgt_tokens.reshape(N, 1)
    out = pl.pallas_call(
        _tc_body,
        grid=(V // CB,),
        in_specs=[
            pl.BlockSpec((N, CB), lambda j: (0, j)),
            pl.BlockSpec((N, 1), lambda j: (0, 0)),
            pl.BlockSpec((NW, L), lambda j: (0, 0)),
        ],
        out_specs=pl.BlockSpec(memory_space=pltpu.SMEM),
        out_shape=jax.ShapeDtypeStruct((1, 1), jnp.float32),
    )(p, tgt2d, scp)
    return out[0, 0]


# speedup vs baseline: 8.0558x; 1.0163x over previous
import functools
import math

import jax
import jax.numpy as jnp
from jax import lax
from jax.experimental import pallas as pl
from jax.experimental.pallas import tpu as pltpu
from jax.experimental.pallas import tpu_sc as plsc

N = 4096
V = 32768
SMOOTH = 0.1
CONF = 1.0 - SMOOTH
R = SMOOTH / (V - 2)
K = (V - 2) * R * math.log(R) + CONF * math.log(CONF)

NC = 2
NS = 16
L = 16
NW = NC * NS
RPW = N // NW

G = 16
RB = 128


def _make_sc_gather():
    mesh = plsc.VectorSubcoreMesh(core_axis_name="c", subcore_axis_name="s")

    @functools.partial(
        pl.kernel,
        mesh=mesh,
        out_type=jax.ShapeDtypeStruct((NW, L), jnp.float32),
        scratch_types=[
            pltpu.SMEM((G,), jnp.int32),
            pltpu.VMEM((RPW,), jnp.float32),
            pltpu.VMEM((G, 8, 128), jnp.float32),
            pltpu.VMEM((L,), jnp.float32),
            pltpu.VMEM((L,), jnp.float32),
            pltpu.SemaphoreType.DMA,
        ],
    )
    def sc_gather(p_hbm, tgt_hbm, out_hbm, tgt_s, tgt_v, tiles_v, mtmp_v,
                  acc_v, sem):
        wid = lax.axis_index("s") * NC + lax.axis_index("c")
        base = wid * RPW
        pltpu.sync_copy(tgt_hbm.at[pl.ds(base, RPW)], tgt_v)
        iota = lax.iota(jnp.int32, L)

        def chunk(g, acc):
            g0 = pl.multiple_of(g * G, G)
            tv = tgt_v[pl.ds(g0, G)]
            for k in range(G):
                tgt_s[k] = tv[k].astype(jnp.int32)
            ts = [tgt_s[k] for k in range(G)]
            copies = []
            for k in range(G):
                t = ts[k]
                r0 = pl.multiple_of(base + g0 + (k & ~7), 8)
                c0 = pl.multiple_of((t >> 7) << 7, 128)
                copies.append(
                    pltpu.async_copy(
                        p_hbm.at[pl.ds(r0, 8), pl.ds(c0, 128)],
                        tiles_v.at[k], sem))
            for c in copies:
                c.wait()
            for k in range(G):
                t = ts[k]
                cl = t & 0x70
                vec = tiles_v[k, k & 7, pl.ds(cl, L)]
                lane = jnp.where(t != 0, t & (L - 1), -1)
                acc = acc + jnp.where(iota == lane,
                                      (R - CONF) * vec + K, 0.0)
            return acc

        acc = lax.fori_loop(0, RPW // G, chunk, jnp.zeros((L,), jnp.float32))
        acc_v[...] = acc
        pltpu.sync_copy(acc_v, out_hbm.at[wid])

    return sc_gather


def _tc_body(p_ref, tgt_ref, scp_ref, out_ref):
    j = pl.program_id(0)
    mask = tgt_ref[...] != 0
    rowsums = jnp.sum(p_ref[...], axis=1, keepdims=True)
    bsum = jnp.sum(jnp.where(mask, rowsums, 0.0))
    col0 = jnp.sum(jnp.where(mask, p_ref[:, 0:1], 0.0))
    part = R * col0 - R * bsum

    @pl.when(j == 0)
    def _():
        out_ref[0, 0] = jnp.sum(scp_ref[...]) + part

    @pl.when(j > 0)
    def _():
        out_ref[0, 0] = out_ref[0, 0] + part


def kernel(predicted_log_probabilities, tgt_tokens):
    p = predicted_log_probabilities
    scp = _make_sc_gather()(p, tgt_tokens.astype(jnp.float32))
    tgt2d = tgt_tokens.reshape(N, 1)
    out = pl.pallas_call(
        _tc_body,
        grid=(N // RB,),
        in_specs=[
            pl.BlockSpec((RB, V), lambda j: (j, 0)),
            pl.BlockSpec((RB, 1), lambda j: (j, 0)),
            pl.BlockSpec((NW, L), lambda j: (0, 0)),
        ],
        out_specs=pl.BlockSpec(memory_space=pltpu.SMEM),
        out_shape=jax.ShapeDtypeStruct((1, 1), jnp.float32),
    )(p, tgt2d, scp)
    return out[0, 0]


# speedup vs baseline: 8.2951x; 1.0297x over previous
import functools
import math

import jax
import jax.numpy as jnp
from jax import lax
from jax.experimental import pallas as pl
from jax.experimental.pallas import tpu as pltpu
from jax.experimental.pallas import tpu_sc as plsc

N = 4096
V = 32768
SMOOTH = 0.1
CONF = 1.0 - SMOOTH
R = SMOOTH / (V - 2)
K = (V - 2) * R * math.log(R) + CONF * math.log(CONF)

NC = 2
NS = 16
L = 16
NW = NC * NS
RPW = N // NW

G = 16
RB = 128


def _make_sc_gather():
    mesh = plsc.VectorSubcoreMesh(core_axis_name="c", subcore_axis_name="s")

    @functools.partial(
        pl.kernel,
        mesh=mesh,
        out_type=jax.ShapeDtypeStruct((NW, L), jnp.float32),
        scratch_types=[
            pltpu.SMEM((G,), jnp.int32),
            pltpu.VMEM((RPW,), jnp.float32),
            pltpu.VMEM((G, 8, 128), jnp.float32),
            pltpu.VMEM((L,), jnp.float32),
            pltpu.VMEM((L,), jnp.float32),
            pltpu.SemaphoreType.DMA,
        ],
    )
    def sc_gather(p_hbm, tgt_hbm, out_hbm, tgt_s, tgt_v, tiles_v, mtmp_v,
                  acc_v, sem):
        wid = lax.axis_index("s") * NC + lax.axis_index("c")
        base = wid * RPW
        pltpu.sync_copy(tgt_hbm.at[pl.ds(base, RPW)], tgt_v)
        iota = lax.iota(jnp.int32, L)

        def chunk(g, acc):
            g0 = pl.multiple_of(g * G, G)
            tv = tgt_v[pl.ds(g0, G)]
            for k in range(G):
                tgt_s[k] = tv[k].astype(jnp.int32)
            ts = [tgt_s[k] for k in range(G)]
            copies = []
            for k in range(G):
                t = ts[k]
                r0 = pl.multiple_of(base + g0 + (k & ~7), 8)
                c0 = pl.multiple_of((t >> 7) << 7, 128)
                copies.append(
                    pltpu.async_copy(
                        p_hbm.at[pl.ds(r0, 8), pl.ds(c0, 128)],
                        tiles_v.at[k], sem))
            for c in copies:
                c.wait()
            for k in range(G):
                t = ts[k]
                cl = t & 0x70
                vec = tiles_v[k, k & 7, pl.ds(cl, L)]
                lane = jnp.where(t != 0, t & (L - 1), -1)
                acc = acc + jnp.where(iota == lane,
                                      (R - CONF) * vec + K, 0.0)
            return acc

        acc = lax.fori_loop(0, RPW // G, chunk, jnp.zeros((L,), jnp.float32))
        acc_v[...] = acc
        pltpu.sync_copy(acc_v, out_hbm.at[wid])

    return sc_gather


def _tc_body(p_ref, tgt_ref, out_ref):
    j = pl.program_id(0)
    mask = tgt_ref[...] != 0
    rowsums = jnp.sum(p_ref[...], axis=1, keepdims=True)
    bsum = jnp.sum(jnp.where(mask, rowsums, 0.0))
    col0 = jnp.sum(jnp.where(mask, p_ref[:, 0:1], 0.0))
    part = R * col0 - R * bsum

    @pl.when(j == 0)
    def _():
        out_ref[0, 0] = part

    @pl.when(j > 0)
    def _():
        out_ref[0, 0] = out_ref[0, 0] + part


def _combine_body(tcp_ref, scp_ref, out_ref):
    out_ref[0, 0] = tcp_ref[0, 0] + jnp.sum(scp_ref[...])


def kernel(predicted_log_probabilities, tgt_tokens):
    p = predicted_log_probabilities
    scp = _make_sc_gather()(p, tgt_tokens.astype(jnp.float32))
    tgt2d = tgt_tokens.reshape(N, 1)
    tcp = pl.pallas_call(
        _tc_body,
        grid=(N // RB,),
        in_specs=[
            pl.BlockSpec((RB, V), lambda j: (j, 0)),
            pl.BlockSpec((RB, 1), lambda j: (j, 0)),
        ],
        out_specs=pl.BlockSpec(memory_space=pltpu.SMEM),
        out_shape=jax.ShapeDtypeStruct((1, 1), jnp.float32),
    )(p, tgt2d)
    out = pl.pallas_call(
        _combine_body,
        in_specs=[
            pl.BlockSpec(memory_space=pltpu.SMEM),
            pl.BlockSpec((NW, L), lambda: (0, 0)),
        ],
        out_specs=pl.BlockSpec(memory_space=pltpu.SMEM),
        out_shape=jax.ShapeDtypeStruct((1, 1), jnp.float32),
    )(tcp, scp)
    return out[0, 0]
